# baseline (speedup 1.0000x reference)
import functools

import jax
import jax.numpy as jnp
from jax import lax
from jax.experimental import pallas as pl
from jax.experimental.pallas import tpu as pltpu
from jax.experimental.pallas import tpu_sc as plsc

B = 16384
L = 20
D = 64
V = 1000000
NC = 2
NS = 16
LANES = 16
NW = NC * NS
BPW = B // NW
CB = 32
NCHUNK = BPW // CB
NPAIR = CB * L
NG = NPAIR // LANES
NIDX = NPAIR // 128


def _sc_body(center_hbm, context_hbm, emb_in_hbm, emb_out_hbm, out_hbm,
             cidx_v, cpidx_v, ccol_v, xidx_v, pidx_v, xcol_v,
             in_rows_v, ctx_rows_v, out_v, sem_in, sem_ctx):
    wid = lax.axis_index("s") * NC + lax.axis_index("c")
    lane = lax.iota(jnp.int32, LANES)

    def chunk_body(ci, carry):
        base_b = wid * BPW + ci * CB
        pltpu.sync_copy(center_hbm.at[pl.ds(base_b, CB)], cidx_v)
        pltpu.sync_copy(context_hbm.at[pl.ds(base_b * L, NPAIR)], xidx_v)
        for j in range(CB // LANES):
            c = cidx_v[pl.ds(j * LANES, LANES)]
            cpidx_v[pl.ds(j * LANES, LANES)] = lax.shift_right_logical(c, 1)
            ccol_v[pl.ds(j * LANES, LANES)] = lax.shift_left(
                lax.bitwise_and(c, 1), 6)

        def prep_body(j, carry2):
            x = xidx_v[pl.ds(j * LANES, LANES)]
            pidx_v[pl.ds(j * LANES, LANES)] = lax.shift_right_logical(x, 1)
            xcol_v[pl.ds(j * LANES, LANES)] = lax.shift_left(
                lax.bitwise_and(x, 1), 6)
            return carry2

        lax.fori_loop(0, NPAIR // LANES, prep_body, 0, unroll=4)

        cp_in = pltpu.async_copy(emb_in_hbm.at[cpidx_v], in_rows_v, sem_in)
        cps = [
            pltpu.async_copy(
                emb_out_hbm.at[pidx_v.at[pl.ds(j * 128, 128)]],
                ctx_rows_v.at[pl.ds(j * 128, 128)],
                sem_ctx,
            )
            for j in range(NIDX)
        ]
        cp_in.wait()
        for cp in cps:
            cp.wait()

        def group_body(g, carry2):
            r_vec = lane + g * LANES
            b_vec = lax.div(r_vec, L)
            in_col = plsc.load_gather(ccol_v, [b_vec])
            ctx_col = xcol_v[pl.ds(g * LANES, LANES)]
            acc = plsc.load_gather(ctx_rows_v, [r_vec, ctx_col]) * \
                plsc.load_gather(in_rows_v, [b_vec, in_col])
            for d in range(1, D):
                vo = plsc.load_gather(ctx_rows_v, [r_vec, ctx_col + d])
                vi = plsc.load_gather(in_rows_v, [b_vec, in_col + d])
                acc = acc + vo * vi
            out_v[pl.ds(g * LANES, LANES)] = acc
            return carry2

        lax.fori_loop(0, NG, group_body, 0, unroll=False)
        pltpu.sync_copy(out_v, out_hbm.at[pl.ds(base_b * L, NPAIR)])
        return carry

    lax.fori_loop(0, NCHUNK, chunk_body, 0, unroll=False)


@functools.partial(jax.jit, static_argnames=())
def _run(center_flat, context_flat, emb_in_r, emb_out_r):
    mesh = plsc.VectorSubcoreMesh(
        core_axis_name="c", subcore_axis_name="s",
        num_cores=NC, num_subcores=NS)
    grid_kernel = pl.kernel(
        _sc_body,
        out_type=jax.ShapeDtypeStruct((B * L,), jnp.float32),
        mesh=mesh,
        scratch_types=[
            pltpu.VMEM((CB,), jnp.int32),
            pltpu.VMEM((CB,), jnp.int32),
            pltpu.VMEM((CB,), jnp.int32),
            pltpu.VMEM((NPAIR,), jnp.int32),
            pltpu.VMEM((NPAIR,), jnp.int32),
            pltpu.VMEM((NPAIR,), jnp.int32),
            pltpu.VMEM((CB, 128), jnp.float32),
            pltpu.VMEM((NPAIR, 128), jnp.float32),
            pltpu.VMEM((NPAIR,), jnp.float32),
            pltpu.SemaphoreType.DMA,
            pltpu.SemaphoreType.DMA,
        ],
        compiler_params=pltpu.CompilerParams(
            needs_layout_passes=False, use_tc_tiling_on_sc=True),
    )
    return grid_kernel(center_flat, context_flat, emb_in_r, emb_out_r)


def kernel(center, context, emb_in, emb_out):
    center_flat = center.reshape(B)
    context_flat = context.reshape(B * L)
    emb_in_r = emb_in.reshape(V // 2, 2 * D)
    emb_out_r = emb_out.reshape(V // 2, 2 * D)
    out_flat = _run(center_flat, context_flat, emb_in_r, emb_out_r)
    return out_flat.reshape(B, L)


# speedup vs baseline: 1.7981x; 1.7981x over previous
import functools

import jax
import jax.numpy as jnp
from jax import lax
from jax.experimental import pallas as pl
from jax.experimental.pallas import tpu as pltpu
from jax.experimental.pallas import tpu_sc as plsc

B = 16384
L = 20
D = 64
V = 1000000
NC = 2
NS = 16
LANES = 16
NW = NC * NS
BPW = B // NW
CB = 32
NCHUNK = BPW // CB
GB = 4
NGRP = CB // GB
QPG = GB * L
NRED = QPG // LANES
NPAIR = CB * L
RB = 16
NB = NPAIR // RB


def _sc_body(center_hbm, context_hbm, emb_in_hbm, emb_out_hbm, out_hbm,
             cidx_v, xidx_v, in_rows_v, ctx_rows_v, tmp_v,
             out_v, sem_in, sem_ctx):
    wid = lax.axis_index("s") * NC + lax.axis_index("c")
    lane = lax.iota(jnp.int32, LANES)
    lane16 = lane * LANES
    zero16 = jnp.zeros((LANES,), jnp.int32)

    def _scalar(vec, t):
        return lax.reduce_max(
            lax.select(lane == t, vec, zero16), axes=(0,))

    def chunk_body(ci, carry):
        base_b = wid * BPW + ci * CB
        pltpu.sync_copy(center_hbm.at[pl.ds(base_b, CB)], cidx_v)
        pltpu.sync_copy(context_hbm.at[pl.ds(base_b * L, NPAIR)], xidx_v)

        def cin_body(j, c2):
            cv = cidx_v[pl.ds(j * LANES, LANES)]
            for t in range(LANES):
                pltpu.async_copy(emb_in_hbm.at[_scalar(cv, t)],
                                 in_rows_v.at[j * LANES + t], sem_in)
            return c2

        lax.fori_loop(0, CB // LANES, cin_body, 0, unroll=False)

        def _issue(g):
            xv = xidx_v[pl.ds(g * RB, RB)]
            for t in range(RB):
                pltpu.async_copy(emb_out_hbm.at[_scalar(xv, t)],
                                 ctx_rows_v.at[g * RB + t], sem_ctx)

        _issue(0)

        def ctx_body(g, c2):
            _issue(g)
            pltpu.make_async_copy(
                emb_out_hbm.at[pl.ds(0, RB)],
                ctx_rows_v.at[pl.ds((g - 1) * RB, RB)],
                sem_ctx).wait()
            return c2

        lax.fori_loop(1, NB, ctx_body, 0, unroll=False)
        pltpu.make_async_copy(
            emb_out_hbm.at[pl.ds(0, RB)],
            ctx_rows_v.at[pl.ds((NB - 1) * RB, RB)],
            sem_ctx).wait()
        pltpu.make_async_copy(
            emb_in_hbm.at[pl.ds(0, CB)], in_rows_v, sem_in).wait()

        def group_body(g4, carry2):
            b0 = g4 * GB
            a = [[in_rows_v[b0 + bb, pl.ds(c * LANES, LANES)]
                  for c in range(D // LANES)] for bb in range(GB)]
            for bb in range(GB):
                for l in range(L):
                    r = (b0 + bb) * L + l
                    q = a[bb][0] * ctx_rows_v[r, pl.ds(0, LANES)]
                    for c in range(1, D // LANES):
                        q = q + a[bb][c] * ctx_rows_v[r, pl.ds(c * LANES, LANES)]
                    tmp_v[pl.ds((bb * L + l) * LANES, LANES)] = q
            for g in range(NRED):
                acc = plsc.load_gather(tmp_v, [lane16 + g * (LANES * LANES)])
                for dd in range(1, LANES):
                    acc = acc + plsc.load_gather(
                        tmp_v, [lane16 + (g * (LANES * LANES) + dd)])
                out_v[pl.ds(g4 * QPG + g * LANES, LANES)] = acc
            return carry2

        lax.fori_loop(0, NGRP, group_body, 0, unroll=False)
        pltpu.sync_copy(out_v, out_hbm.at[pl.ds(base_b * L, NPAIR)])
        return carry

    lax.fori_loop(0, NCHUNK, chunk_body, 0, unroll=False)


@functools.partial(jax.jit, static_argnames=())
def _run(center_flat, context_flat, emb_in, emb_out):
    mesh = plsc.VectorSubcoreMesh(
        core_axis_name="c", subcore_axis_name="s",
        num_cores=NC, num_subcores=NS)
    grid_kernel = pl.kernel(
        _sc_body,
        out_type=jax.ShapeDtypeStruct((B * L,), jnp.float32),
        mesh=mesh,
        scratch_types=[
            pltpu.VMEM((CB,), jnp.int32),
            pltpu.VMEM((NPAIR,), jnp.int32),
            pltpu.VMEM((CB, D), jnp.float32),
            pltpu.VMEM((NPAIR, D), jnp.float32),
            pltpu.VMEM((QPG * LANES,), jnp.float32),
            pltpu.VMEM((NPAIR,), jnp.float32),
            pltpu.SemaphoreType.DMA,
            pltpu.SemaphoreType.DMA,
        ],
        compiler_params=pltpu.CompilerParams(
            needs_layout_passes=False, use_tc_tiling_on_sc=True),
    )
    return grid_kernel(center_flat, context_flat, emb_in, emb_out)


def kernel(center, context, emb_in, emb_out):
    center_flat = center.reshape(B)
    context_flat = context.reshape(B * L)
    out_flat = _run(center_flat, context_flat, emb_in, emb_out)
    return out_flat.reshape(B, L)
